# baseline (speedup 1.0000x reference)
import functools

import jax
import jax.numpy as jnp
from jax.experimental import pallas as pl

N_NODES = 1024
N_NEIGH = 32
T = 8
B = 16
F_IN = 128
EMB = 128
HID = 128
NCLS = 10

ROW_BLK = 128
INV_DEG = 1.0 / (N_NEIGH + 1)


def _build_a_body(adj_ref, a_ref):
    row0 = pl.program_id(0) * ROW_BLK
    col = jax.lax.broadcasted_iota(jnp.int32, (ROW_BLK, N_NODES), 1)
    row = jax.lax.broadcasted_iota(jnp.int32, (ROW_BLK, N_NODES), 0) + row0
    acc = (col == row).astype(jnp.float32)
    for j in range(N_NEIGH):
        acc += (col == adj_ref[:, j][:, None]).astype(jnp.float32)
    a_ref[...] = acc.astype(jnp.bfloat16)


def _gcn_body(x_ref, a_ref, w1_ref, b1_ref, w2_ref, b2_ref, out_ref):
    x = x_ref[0, 0].astype(jnp.bfloat16)
    a = a_ref[...]
    y = jnp.dot(x, w1_ref[...], preferred_element_type=jnp.float32)
    z = jnp.dot(a, y.astype(jnp.bfloat16), preferred_element_type=jnp.float32)
    h1 = jnp.maximum(z * INV_DEG + b1_ref[...], 0.0)
    y2 = jnp.dot(h1.astype(jnp.bfloat16), w2_ref[...], preferred_element_type=jnp.float32)
    z2 = jnp.dot(a, y2.astype(jnp.bfloat16), preferred_element_type=jnp.float32)
    h2 = jnp.maximum(z2 * INV_DEG + b2_ref[...], 0.0)
    out_ref[0, 0, 0] = jnp.sum(h2, axis=0)


def _lstm_body(seq_ref, wih_ref, whh_ref, b_ref, wc_ref, bc_ref, out_ref):
    h = jnp.zeros((B, HID), dtype=jnp.float32)
    c = jnp.zeros((B, HID), dtype=jnp.float32)
    for t in range(T):
        x = seq_ref[t]
        gates = (jnp.dot(x, wih_ref[...], preferred_element_type=jnp.float32)
                 + jnp.dot(h, whh_ref[...], preferred_element_type=jnp.float32)
                 + b_ref[...])
        i = jax.nn.sigmoid(gates[:, 0 * HID:1 * HID])
        f = jax.nn.sigmoid(gates[:, 1 * HID:2 * HID])
        g = jnp.tanh(gates[:, 2 * HID:3 * HID])
        o = jax.nn.sigmoid(gates[:, 3 * HID:4 * HID])
        c = f * c + i * g
        h = o * jnp.tanh(c)
    hr = jnp.maximum(h, 0.0)
    logits = jnp.dot(hr, wc_ref[...], preferred_element_type=jnp.float32) + bc_ref[...]
    logits = logits - jnp.max(logits, axis=1, keepdims=True)
    e = jnp.exp(logits)
    out_ref[...] = e / jnp.sum(e, axis=1, keepdims=True)


@jax.jit
def kernel(node_features, adjacent_mappings, W1, b1, W2, b2, W_ih, W_hh, b_ih, b_hh, Wc, bc):
    adj = adjacent_mappings.astype(jnp.int32)

    a_mat = pl.pallas_call(
        _build_a_body,
        grid=(N_NODES // ROW_BLK,),
        in_specs=[pl.BlockSpec((ROW_BLK, N_NEIGH), lambda i: (i, 0))],
        out_specs=pl.BlockSpec((ROW_BLK, N_NODES), lambda i: (i, 0)),
        out_shape=jax.ShapeDtypeStruct((N_NODES, N_NODES), jnp.bfloat16),
    )(adj)

    seq = pl.pallas_call(
        _gcn_body,
        grid=(T, B),
        in_specs=[
            pl.BlockSpec((1, 1, N_NODES, F_IN), lambda t, b: (t, b, 0, 0)),
            pl.BlockSpec((N_NODES, N_NODES), lambda t, b: (0, 0)),
            pl.BlockSpec((F_IN, F_IN), lambda t, b: (0, 0)),
            pl.BlockSpec((1, F_IN), lambda t, b: (0, 0)),
            pl.BlockSpec((F_IN, EMB), lambda t, b: (0, 0)),
            pl.BlockSpec((1, EMB), lambda t, b: (0, 0)),
        ],
        out_specs=pl.BlockSpec((1, 1, 1, EMB), lambda t, b: (t, b, 0, 0)),
        out_shape=jax.ShapeDtypeStruct((T, B, 1, EMB), jnp.float32),
    )(node_features, a_mat, W1.astype(jnp.bfloat16), b1.reshape(1, F_IN),
      W2.astype(jnp.bfloat16), b2.reshape(1, EMB))
    seq = seq.reshape(T, B, EMB)

    out = pl.pallas_call(
        _lstm_body,
        in_specs=[
            pl.BlockSpec((T, B, EMB), lambda: (0, 0, 0)),
            pl.BlockSpec((EMB, 4 * HID), lambda: (0, 0)),
            pl.BlockSpec((HID, 4 * HID), lambda: (0, 0)),
            pl.BlockSpec((1, 4 * HID), lambda: (0, 0)),
            pl.BlockSpec((HID, NCLS), lambda: (0, 0)),
            pl.BlockSpec((1, NCLS), lambda: (0, 0)),
        ],
        out_specs=pl.BlockSpec((B, NCLS), lambda: (0, 0)),
        out_shape=jax.ShapeDtypeStruct((B, NCLS), jnp.float32),
    )(seq, W_ih.T, W_hh.T, (b_ih + b_hh).reshape(1, 4 * HID), Wc.T, bc.reshape(1, NCLS))

    return out


# speedup vs baseline: 2.9369x; 2.9369x over previous
import functools

import jax
import jax.numpy as jnp
from jax.experimental import pallas as pl

N_NODES = 1024
N_NEIGH = 32
T = 8
B = 16
F_IN = 128
EMB = 128
HID = 128
NCLS = 10

ROW_BLK = 128
INV_DEG = 1.0 / (N_NEIGH + 1)


def _build_a_body(adj_ref, a_ref):
    row0 = pl.program_id(0) * ROW_BLK
    col = jax.lax.broadcasted_iota(jnp.int32, (ROW_BLK, N_NODES), 1)
    row = jax.lax.broadcasted_iota(jnp.int32, (ROW_BLK, N_NODES), 0) + row0
    acc = (col == row).astype(jnp.float32)
    for j in range(N_NEIGH):
        acc += (col == adj_ref[:, j][:, None]).astype(jnp.float32)
    a_ref[...] = acc.astype(jnp.bfloat16)


BT = 8


def _gcn_body(x_ref, a_ref, w1_ref, b1t_ref, w2_ref, b2t_ref, out_ref):
    a = a_ref[...]
    y = jnp.concatenate(
        [jnp.dot(x_ref[0, 0, b].astype(jnp.bfloat16), w1_ref[...],
                 preferred_element_type=jnp.float32).astype(jnp.bfloat16)
         for b in range(BT)],
        axis=1)
    z = jnp.dot(a, y, preferred_element_type=jnp.float32)
    h1 = (jnp.maximum(z * INV_DEG + b1t_ref[...], 0.0)).astype(jnp.bfloat16)
    y2 = jnp.concatenate(
        [jnp.dot(h1[:, b * F_IN:(b + 1) * F_IN], w2_ref[...],
                 preferred_element_type=jnp.float32).astype(jnp.bfloat16)
         for b in range(BT)],
        axis=1)
    z2 = jnp.dot(a, y2, preferred_element_type=jnp.float32)
    h2 = jnp.maximum(z2 * INV_DEG + b2t_ref[...], 0.0)
    s = jnp.sum(h2, axis=0)
    out_ref[0, 0] = s.reshape(BT, EMB)


def _lstm_body(seq_ref, wih_ref, whh_ref, b_ref, wc_ref, bc_ref, out_ref):
    h = jnp.zeros((B, HID), dtype=jnp.float32)
    c = jnp.zeros((B, HID), dtype=jnp.float32)
    for t in range(T):
        x = seq_ref[t]
        gates = (jnp.dot(x, wih_ref[...], preferred_element_type=jnp.float32)
                 + jnp.dot(h, whh_ref[...], preferred_element_type=jnp.float32)
                 + b_ref[...])
        i = jax.nn.sigmoid(gates[:, 0 * HID:1 * HID])
        f = jax.nn.sigmoid(gates[:, 1 * HID:2 * HID])
        g = jnp.tanh(gates[:, 2 * HID:3 * HID])
        o = jax.nn.sigmoid(gates[:, 3 * HID:4 * HID])
        c = f * c + i * g
        h = o * jnp.tanh(c)
    hr = jnp.maximum(h, 0.0)
    logits = jnp.dot(hr, wc_ref[...], preferred_element_type=jnp.float32) + bc_ref[...]
    logits = logits - jnp.max(logits, axis=1, keepdims=True)
    e = jnp.exp(logits)
    out_ref[...] = e / jnp.sum(e, axis=1, keepdims=True)


@jax.jit
def kernel(node_features, adjacent_mappings, W1, b1, W2, b2, W_ih, W_hh, b_ih, b_hh, Wc, bc):
    adj = adjacent_mappings.astype(jnp.int32)

    a_mat = pl.pallas_call(
        _build_a_body,
        grid=(N_NODES // ROW_BLK,),
        in_specs=[pl.BlockSpec((ROW_BLK, N_NEIGH), lambda i: (i, 0))],
        out_specs=pl.BlockSpec((ROW_BLK, N_NODES), lambda i: (i, 0)),
        out_shape=jax.ShapeDtypeStruct((N_NODES, N_NODES), jnp.bfloat16),
    )(adj)

    nf = node_features.reshape(T, B // BT, BT, N_NODES, F_IN)
    seq = pl.pallas_call(
        _gcn_body,
        grid=(T, B // BT),
        in_specs=[
            pl.BlockSpec((1, 1, BT, N_NODES, F_IN), lambda t, g: (t, g, 0, 0, 0)),
            pl.BlockSpec((N_NODES, N_NODES), lambda t, g: (0, 0)),
            pl.BlockSpec((F_IN, F_IN), lambda t, g: (0, 0)),
            pl.BlockSpec((1, BT * F_IN), lambda t, g: (0, 0)),
            pl.BlockSpec((F_IN, EMB), lambda t, g: (0, 0)),
            pl.BlockSpec((1, BT * EMB), lambda t, g: (0, 0)),
        ],
        out_specs=pl.BlockSpec((1, 1, BT, EMB), lambda t, g: (t, g, 0, 0)),
        out_shape=jax.ShapeDtypeStruct((T, B // BT, BT, EMB), jnp.float32),
    )(nf, a_mat, W1.astype(jnp.bfloat16), jnp.tile(b1, BT).reshape(1, BT * F_IN),
      W2.astype(jnp.bfloat16), jnp.tile(b2, BT).reshape(1, BT * EMB))
    seq = seq.reshape(T, B, EMB)

    out = pl.pallas_call(
        _lstm_body,
        in_specs=[
            pl.BlockSpec((T, B, EMB), lambda: (0, 0, 0)),
            pl.BlockSpec((EMB, 4 * HID), lambda: (0, 0)),
            pl.BlockSpec((HID, 4 * HID), lambda: (0, 0)),
            pl.BlockSpec((1, 4 * HID), lambda: (0, 0)),
            pl.BlockSpec((HID, NCLS), lambda: (0, 0)),
            pl.BlockSpec((1, NCLS), lambda: (0, 0)),
        ],
        out_specs=pl.BlockSpec((B, NCLS), lambda: (0, 0)),
        out_shape=jax.ShapeDtypeStruct((B, NCLS), jnp.float32),
    )(seq, W_ih.T, W_hh.T, (b_ih + b_hh).reshape(1, 4 * HID), Wc.T, bc.reshape(1, NCLS))

    return out


# speedup vs baseline: 2.9991x; 1.0212x over previous
import jax
import jax.numpy as jnp
from jax.experimental import pallas as pl
from jax.experimental.pallas import tpu as pltpu

N_NODES = 1024
N_NEIGH = 32
T = 8
B = 16
F_IN = 128
EMB = 128
HID = 128
NCLS = 10

ROW_BLK = 256
INV_DEG = 1.0 / (N_NEIGH + 1)


BT = 8


def _gcn_body(adj_ref, x_ref, w1_ref, b1t_ref, w2_ref, b2t_ref, out_ref, a_scr):
    @pl.when((pl.program_id(0) == 0) & (pl.program_id(1) == 0))
    def _build_a():
        for blk in range(N_NODES // ROW_BLK):
            row0 = blk * ROW_BLK
            col = jax.lax.broadcasted_iota(jnp.int32, (ROW_BLK, N_NODES), 1)
            row = jax.lax.broadcasted_iota(jnp.int32, (ROW_BLK, N_NODES), 0) + row0
            acc = (col == row).astype(jnp.float32)
            for j in range(N_NEIGH):
                acc += (col == adj_ref[row0:row0 + ROW_BLK, j][:, None]).astype(jnp.float32)
            a_scr[row0:row0 + ROW_BLK, :] = acc.astype(jnp.bfloat16)

    a = a_scr[...]
    y = jnp.concatenate(
        [jnp.dot(x_ref[0, 0, b].astype(jnp.bfloat16), w1_ref[...],
                 preferred_element_type=jnp.float32).astype(jnp.bfloat16)
         for b in range(BT)],
        axis=1)
    z = jnp.dot(a, y, preferred_element_type=jnp.float32)
    h1 = (jnp.maximum(z * INV_DEG + b1t_ref[...], 0.0)).astype(jnp.bfloat16)
    y2 = jnp.concatenate(
        [jnp.dot(h1[:, b * F_IN:(b + 1) * F_IN], w2_ref[...],
                 preferred_element_type=jnp.float32).astype(jnp.bfloat16)
         for b in range(BT)],
        axis=1)
    z2 = jnp.dot(a, y2, preferred_element_type=jnp.float32)
    h2 = jnp.maximum(z2 * INV_DEG + b2t_ref[...], 0.0)
    s = jnp.sum(h2, axis=0)
    out_ref[0, 0] = s.reshape(BT, EMB)


def _lstm_body(seq_ref, wih_ref, whh_ref, b_ref, wc_ref, bc_ref, out_ref):
    h = jnp.zeros((B, HID), dtype=jnp.float32)
    c = jnp.zeros((B, HID), dtype=jnp.float32)
    for t in range(T):
        x = seq_ref[t]
        gates = (jnp.dot(x, wih_ref[...], preferred_element_type=jnp.float32)
                 + jnp.dot(h, whh_ref[...], preferred_element_type=jnp.float32)
                 + b_ref[...])
        i = jax.nn.sigmoid(gates[:, 0 * HID:1 * HID])
        f = jax.nn.sigmoid(gates[:, 1 * HID:2 * HID])
        g = jnp.tanh(gates[:, 2 * HID:3 * HID])
        o = jax.nn.sigmoid(gates[:, 3 * HID:4 * HID])
        c = f * c + i * g
        h = o * jnp.tanh(c)
    hr = jnp.maximum(h, 0.0)
    logits = jnp.dot(hr, wc_ref[...], preferred_element_type=jnp.float32) + bc_ref[...]
    logits = logits - jnp.max(logits, axis=1, keepdims=True)
    e = jnp.exp(logits)
    out_ref[...] = e / jnp.sum(e, axis=1, keepdims=True)


@jax.jit
def kernel(node_features, adjacent_mappings, W1, b1, W2, b2, W_ih, W_hh, b_ih, b_hh, Wc, bc):
    adj = adjacent_mappings.astype(jnp.int32)

    nf = node_features.reshape(T, B // BT, BT, N_NODES, F_IN)
    seq = pl.pallas_call(
        _gcn_body,
        grid=(T, B // BT),
        in_specs=[
            pl.BlockSpec((N_NODES, N_NEIGH), lambda t, g: (0, 0)),
            pl.BlockSpec((1, 1, BT, N_NODES, F_IN), lambda t, g: (t, g, 0, 0, 0)),
            pl.BlockSpec((F_IN, F_IN), lambda t, g: (0, 0)),
            pl.BlockSpec((1, BT * F_IN), lambda t, g: (0, 0)),
            pl.BlockSpec((F_IN, EMB), lambda t, g: (0, 0)),
            pl.BlockSpec((1, BT * EMB), lambda t, g: (0, 0)),
        ],
        out_specs=pl.BlockSpec((1, 1, BT, EMB), lambda t, g: (t, g, 0, 0)),
        out_shape=jax.ShapeDtypeStruct((T, B // BT, BT, EMB), jnp.float32),
        scratch_shapes=[pltpu.VMEM((N_NODES, N_NODES), jnp.bfloat16)],
    )(adj, nf, W1.astype(jnp.bfloat16),
      jnp.tile(b1, BT).reshape(1, BT * F_IN),
      W2.astype(jnp.bfloat16), jnp.tile(b2, BT).reshape(1, BT * EMB))
    seq = seq.reshape(T, B, EMB)

    out = pl.pallas_call(
        _lstm_body,
        in_specs=[
            pl.BlockSpec((T, B, EMB), lambda: (0, 0, 0)),
            pl.BlockSpec((EMB, 4 * HID), lambda: (0, 0)),
            pl.BlockSpec((HID, 4 * HID), lambda: (0, 0)),
            pl.BlockSpec((1, 4 * HID), lambda: (0, 0)),
            pl.BlockSpec((HID, NCLS), lambda: (0, 0)),
            pl.BlockSpec((1, NCLS), lambda: (0, 0)),
        ],
        out_specs=pl.BlockSpec((B, NCLS), lambda: (0, 0)),
        out_shape=jax.ShapeDtypeStruct((B, NCLS), jnp.float32),
    )(seq, W_ih.T, W_hh.T, (b_ih + b_hh).reshape(1, 4 * HID), Wc.T, bc.reshape(1, NCLS))

    return out


# speedup vs baseline: 3.2118x; 1.0709x over previous
import jax
import jax.numpy as jnp
from jax.experimental import pallas as pl
from jax.experimental.pallas import tpu as pltpu

N_NODES = 1024
N_NEIGH = 32
T = 8
B = 16
F_IN = 128
EMB = 128
HID = 128
NCLS = 10

ROW_BLK = 256
INV_DEG = 1.0 / (N_NEIGH + 1)
BT = 8
NG = B // BT


def _body(adj_ref, x_ref, w1_ref, b1_ref, w2_ref, b2_ref,
          wih_ref, whh_ref, bg_ref, wc_ref, bc_ref,
          out_ref, a_scr, seq_scr):
    t = pl.program_id(0)
    g = pl.program_id(1)

    @pl.when((t == 0) & (g == 0))
    def _build_a():
        for blk in range(N_NODES // ROW_BLK):
            row0 = blk * ROW_BLK
            col = jax.lax.broadcasted_iota(jnp.int32, (ROW_BLK, N_NODES), 1)
            row = jax.lax.broadcasted_iota(jnp.int32, (ROW_BLK, N_NODES), 0) + row0
            acc = (col == row).astype(jnp.float32)
            for j in range(N_NEIGH):
                acc += (col == adj_ref[row0:row0 + ROW_BLK, j][:, None]).astype(jnp.float32)
            a_scr[row0:row0 + ROW_BLK, :] = acc.astype(jnp.bfloat16)

    a = a_scr[...]
    w1 = w1_ref[...].astype(jnp.bfloat16)
    w2 = w2_ref[...].astype(jnp.bfloat16)
    b1t = jnp.concatenate([b1_ref[...]] * BT, axis=1)
    b2t = jnp.concatenate([b2_ref[...]] * BT, axis=1)
    y = jnp.concatenate(
        [jnp.dot(x_ref[0, 0, b].astype(jnp.bfloat16), w1,
                 preferred_element_type=jnp.float32).astype(jnp.bfloat16)
         for b in range(BT)],
        axis=1)
    z = jnp.dot(a, y, preferred_element_type=jnp.float32)
    h1 = (jnp.maximum(z * INV_DEG + b1t, 0.0)).astype(jnp.bfloat16)
    y2 = jnp.concatenate(
        [jnp.dot(h1[:, b * F_IN:(b + 1) * F_IN], w2,
                 preferred_element_type=jnp.float32).astype(jnp.bfloat16)
         for b in range(BT)],
        axis=1)
    z2 = jnp.dot(a, y2, preferred_element_type=jnp.float32)
    h2 = jnp.maximum(z2 * INV_DEG + b2t, 0.0)
    s = jnp.sum(h2, axis=0)
    seq_scr[t, pl.ds(g * BT, BT), :] = s.reshape(BT, EMB)

    @pl.when((t == T - 1) & (g == NG - 1))
    def _lstm_cls():
        bg = bg_ref[...]
        h = jnp.zeros((B, HID), dtype=jnp.float32)
        c = jnp.zeros((B, HID), dtype=jnp.float32)
        for tt in range(T):
            x = seq_scr[tt]
            gates = (jax.lax.dot_general(x, wih_ref[...], (((1,), (1,)), ((), ())),
                                         preferred_element_type=jnp.float32)
                     + jax.lax.dot_general(h, whh_ref[...], (((1,), (1,)), ((), ())),
                                           preferred_element_type=jnp.float32)
                     + bg)
            i = jax.nn.sigmoid(gates[:, 0 * HID:1 * HID])
            f = jax.nn.sigmoid(gates[:, 1 * HID:2 * HID])
            gg = jnp.tanh(gates[:, 2 * HID:3 * HID])
            o = jax.nn.sigmoid(gates[:, 3 * HID:4 * HID])
            c = f * c + i * gg
            h = o * jnp.tanh(c)
        hr = jnp.maximum(h, 0.0)
        logits = jax.lax.dot_general(hr, wc_ref[...], (((1,), (1,)), ((), ())),
                                     preferred_element_type=jnp.float32) + bc_ref[...]
        logits = logits - jnp.max(logits, axis=1, keepdims=True)
        e = jnp.exp(logits)
        out_ref[...] = e / jnp.sum(e, axis=1, keepdims=True)


@jax.jit
def kernel(node_features, adjacent_mappings, W1, b1, W2, b2, W_ih, W_hh, b_ih, b_hh, Wc, bc):
    adj = adjacent_mappings.astype(jnp.int32)
    nf = node_features.reshape(T, NG, BT, N_NODES, F_IN)

    out = pl.pallas_call(
        _body,
        grid=(T, NG),
        in_specs=[
            pl.BlockSpec((N_NODES, N_NEIGH), lambda t, g: (0, 0)),
            pl.BlockSpec((1, 1, BT, N_NODES, F_IN), lambda t, g: (t, g, 0, 0, 0)),
            pl.BlockSpec((F_IN, F_IN), lambda t, g: (0, 0)),
            pl.BlockSpec((1, F_IN), lambda t, g: (0, 0)),
            pl.BlockSpec((F_IN, EMB), lambda t, g: (0, 0)),
            pl.BlockSpec((1, EMB), lambda t, g: (0, 0)),
            pl.BlockSpec((4 * HID, EMB), lambda t, g: (0, 0)),
            pl.BlockSpec((4 * HID, HID), lambda t, g: (0, 0)),
            pl.BlockSpec((1, 4 * HID), lambda t, g: (0, 0)),
            pl.BlockSpec((NCLS, HID), lambda t, g: (0, 0)),
            pl.BlockSpec((1, NCLS), lambda t, g: (0, 0)),
        ],
        out_specs=pl.BlockSpec((B, NCLS), lambda t, g: (0, 0)),
        out_shape=jax.ShapeDtypeStruct((B, NCLS), jnp.float32),
        scratch_shapes=[pltpu.VMEM((N_NODES, N_NODES), jnp.bfloat16),
                        pltpu.VMEM((T, B, EMB), jnp.float32)],
    )(adj, nf, W1, b1.reshape(1, F_IN), W2, b2.reshape(1, EMB),
      W_ih, W_hh, (b_ih + b_hh).reshape(1, 4 * HID), Wc, bc.reshape(1, NCLS))

    return out
